# baseline (speedup 1.0000x reference)
import jax
import jax.numpy as jnp
from jax.experimental import pallas as pl
from jax.experimental.pallas import tpu as pltpu
from jax.experimental.pallas import tpu_sc as plsc

_N_ROWS = 16384
_N_COLS = 2048
_OUT_ROWS = _N_ROWS // 2

_SPLIT = 6144

_TC_BLK = 512
_SC_BLK_R = 8


def _tc_mean_body(x1_ref, x2_ref, o_ref):
    o_ref[...] = (x1_ref[...] + x2_ref[...]) * 0.5


def _tc_part(x_cat):
    n_blocks = _SPLIT // _TC_BLK
    half_off = _OUT_ROWS // _TC_BLK
    return pl.pallas_call(
        _tc_mean_body,
        grid=(n_blocks,),
        in_specs=[
            pl.BlockSpec((_TC_BLK, _N_COLS), lambda i: (i, 0)),
            pl.BlockSpec((_TC_BLK, _N_COLS), lambda i: (i + half_off, 0)),
        ],
        out_specs=pl.BlockSpec((_TC_BLK, _N_COLS), lambda i: (i, 0)),
        out_shape=jax.ShapeDtypeStruct((_SPLIT, _N_COLS), x_cat.dtype),
        compiler_params=pltpu.CompilerParams(
            dimension_semantics=("arbitrary",),
        ),
    )(x_cat, x_cat)


def _sc_part(x_cat):
    sc_rows = _OUT_ROWS - _SPLIT
    n_blocks = sc_rows // _SC_BLK_R
    off1 = _SPLIT // _SC_BLK_R
    off2 = (_OUT_ROWS + _SPLIT) // _SC_BLK_R
    mesh = plsc.VectorSubcoreMesh(core_axis_name="core", subcore_axis_name="subcore")

    @pl.kernel(
        out_type=jax.ShapeDtypeStruct((sc_rows, _N_COLS), jnp.float32),
        mesh=mesh,
        compiler_params=pltpu.CompilerParams(needs_layout_passes=False),
    )
    def run(x_hbm, y_hbm, o_hbm):
        def body(x1_v, x2_v, o_v):
            for r in range(_SC_BLK_R):
                @plsc.parallel_loop(0, _N_COLS, 16, unroll=8)
                def _(c):
                    s = pl.ds(c, 16)
                    o_v[r, s] = (x1_v[r, s] + x2_v[r, s]) * 0.5

        pltpu.emit_pipeline(
            body,
            grid=(n_blocks,),
            in_specs=[
                pl.BlockSpec((_SC_BLK_R, _N_COLS), lambda i: (i + off1, 0)),
                pl.BlockSpec((_SC_BLK_R, _N_COLS), lambda i: (i + off2, 0)),
            ],
            out_specs=[pl.BlockSpec((_SC_BLK_R, _N_COLS), lambda i: (i, 0))],
            core_axis_name=("core", "subcore"),
            dimension_semantics=(pltpu.PARALLEL,),
        )(x_hbm, y_hbm, o_hbm)

    return run(x_cat, x_cat)


def kernel(x_cat):
    top = _tc_part(x_cat)
    bottom = _sc_part(x_cat)
    return jnp.concatenate([top, bottom], axis=0)


# speedup vs baseline: 1.9493x; 1.9493x over previous
import jax
import jax.numpy as jnp
from jax.experimental import pallas as pl
from jax.experimental.pallas import tpu as pltpu

_BLK = 512


def _mean_kernel(x1_ref, x2_ref, o_ref):
    o_ref[...] = (x1_ref[...] + x2_ref[...]) * 0.5


def kernel(x_cat):
    n_rows, n_cols = x_cat.shape
    x_len = n_rows // 2
    n_blocks = x_len // _BLK
    return pl.pallas_call(
        _mean_kernel,
        grid=(n_blocks,),
        in_specs=[
            pl.BlockSpec((_BLK, n_cols), lambda i: (i, 0)),
            pl.BlockSpec(
                (_BLK, n_cols),
                lambda i, nb=n_blocks: (i + nb, 0),
            ),
        ],
        out_specs=pl.BlockSpec((_BLK, n_cols), lambda i: (i, 0)),
        out_shape=jax.ShapeDtypeStruct((x_len, n_cols), x_cat.dtype),
        compiler_params=pltpu.CompilerParams(
            dimension_semantics=("parallel",),
        ),
    )(x_cat, x_cat)


# speedup vs baseline: 1.9527x; 1.0018x over previous
import jax
import jax.numpy as jnp
from jax.experimental import pallas as pl
from jax.experimental.pallas import tpu as pltpu

_BLK = 512
_SUB = 256


def _mean_kernel(x1a_ref, x1b_ref, x2a_ref, x2b_ref, o_ref):
    o_ref[:_SUB, :] = (x1a_ref[...] + x2a_ref[...]) * 0.5
    o_ref[_SUB:, :] = (x1b_ref[...] + x2b_ref[...]) * 0.5


def kernel(x_cat):
    n_rows, n_cols = x_cat.shape
    x_len = n_rows // 2
    n_blocks = x_len // _BLK
    nb2 = x_len // _SUB
    return pl.pallas_call(
        _mean_kernel,
        grid=(n_blocks,),
        in_specs=[
            pl.BlockSpec((_SUB, n_cols), lambda i: (2 * i, 0)),
            pl.BlockSpec((_SUB, n_cols), lambda i: (2 * i + 1, 0)),
            pl.BlockSpec((_SUB, n_cols), lambda i, nb=nb2: (2 * i + nb, 0)),
            pl.BlockSpec((_SUB, n_cols), lambda i, nb=nb2: (2 * i + 1 + nb, 0)),
        ],
        out_specs=pl.BlockSpec((_BLK, n_cols), lambda i: (i, 0)),
        out_shape=jax.ShapeDtypeStruct((x_len, n_cols), x_cat.dtype),
        compiler_params=pltpu.CompilerParams(
            dimension_semantics=("arbitrary",),
        ),
    )(x_cat, x_cat, x_cat, x_cat)
